# Initial kernel scaffold; baseline (speedup 1.0000x reference)
import jax
import jax.numpy as jnp
from jax.experimental import pallas as pl


def kernel(pos, norm, edge_index, W1, u1, c1, b1, W2, u2, c2, b2, W3, u3, c3, b3, W4, u4, c4, b4, Wl, bl):
    raise NotImplementedError("write your pallas kernel here")


# speedup vs baseline: 14.1578x; 14.1578x over previous
import functools

import jax
import jax.numpy as jnp
from jax import lax
from jax.experimental import pallas as pl
from jax.experimental.pallas import tpu as pltpu
from jax.experimental.pallas import tpu_sc as plsc

NC = 2
NS = 16
NW = NC * NS
CH = 128


@functools.partial(jax.jit, static_argnums=(3,))
def _sc_segment_sum(x, srcr, dstr, n_pad):
    ncw = srcr.shape[1]
    d = x.shape[1]
    rpt = n_pad // NS

    mesh = plsc.VectorSubcoreMesh(core_axis_name="c", subcore_axis_name="s",
                                  num_cores=NC, num_subcores=NS)

    @functools.partial(
        pl.kernel,
        out_type=jax.ShapeDtypeStruct((NC, n_pad, d), jnp.float32),
        mesh=mesh,
        scratch_types=[
            pltpu.VMEM((ncw, CH), jnp.int32),
            pltpu.VMEM((ncw, CH), jnp.int32),
            pltpu.VMEM((CH, d), jnp.float32),
            pltpu.VMEM_SHARED((n_pad, d), jnp.float32),
            pltpu.SemaphoreType.DMA,
        ],
        compiler_params=pltpu.CompilerParams(
            use_tc_tiling_on_sc=False if d < 128 else None),
    )
    def seg(x_hbm, src_hbm, dst_hbm, out_hbm, src_v, dst_v, rows_v, acc_sh, sem):
        c = lax.axis_index("c")
        s = lax.axis_index("s")
        w = c * NS + s

        zv = jnp.zeros((16,), jnp.float32)

        def zrow(i, _):
            for j in range(d // 16):
                rows_v[i, pl.ds(j * 16, 16)] = zv
            return 0

        lax.fori_loop(0, CH, zrow, 0)
        n_full = rpt // CH
        rem = rpt - n_full * CH
        for k in range(n_full):
            pltpu.sync_copy(rows_v, acc_sh.at[pl.ds(s * rpt + k * CH, CH)])
        if rem:
            pltpu.sync_copy(rows_v.at[pl.ds(0, rem)],
                            acc_sh.at[pl.ds(s * rpt + n_full * CH, rem)])
        plsc.subcore_barrier()

        pltpu.sync_copy(src_hbm.at[w], src_v)
        pltpu.sync_copy(dst_hbm.at[w], dst_v)

        def body(j, _):
            pltpu.async_copy(x_hbm.at[src_v.at[j]], rows_v, sem).wait()
            pltpu.sync_copy(rows_v, acc_sh.at[dst_v.at[j]], add=True)
            return 0

        lax.fori_loop(0, ncw, body, 0)
        plsc.subcore_barrier()

        pltpu.sync_copy(acc_sh.at[pl.ds(s * rpt, rpt)],
                        out_hbm.at[c, pl.ds(s * rpt, rpt)])

    return seg(x, srcr, dstr)


_DOT = dict(preferred_element_type=jnp.float32, precision=lax.Precision.HIGHEST)


def _bf_rne(x):
    u = lax.bitcast_convert_type(x, jnp.uint32)
    r = ((u >> 16) & 1) + jnp.uint32(0x7FFF)
    return lax.bitcast_convert_type((u + r) & jnp.uint32(0xFFFF0000),
                                    jnp.float32)


def _l1_body(p_ref, w_ref, e6_ref, b_ref, x_ref, inv_ref):
    p = p_ref[0] + p_ref[1]
    y = jnp.dot(p, w_ref[...], **_DOT)
    cnt = jnp.dot(p, e6_ref[...], **_DOT)
    inv = 1.0 / jnp.maximum(cnt, 1.0)
    x_ref[...] = _bf_rne(jnp.maximum(y * inv + b_ref[...], 0.0))
    inv_ref[...] = inv


def _lmid_body(p_ref, inv_ref, w_ref, b_ref, x_ref):
    agg = (p_ref[0] + p_ref[1]) * inv_ref[...]
    y = jnp.dot(agg, w_ref[...], **_DOT) + b_ref[...]
    x_ref[...] = _bf_rne(jnp.maximum(y, 0.0))


def _l4_body(p_ref, inv_ref, w_ref, b_ref, wl_ref, bl_ref, o_ref):
    agg = (p_ref[0] + p_ref[1]) * inv_ref[...]
    x4 = jnp.maximum(jnp.dot(agg, w_ref[...], **_DOT) + b_ref[...], 0.0)
    o_ref[...] = jnp.dot(_bf_rne(x4), wl_ref[...], **_DOT) + bl_ref[...]


def _tc_call(body, partials, extras, n_out, grid_n=4):
    n_pad, din = partials.shape[1], partials.shape[2]
    br = n_pad // grid_n
    in_specs = [pl.BlockSpec((NC, br, din), lambda i: (0, i, 0))]
    for e in extras:
        if e.shape[0] == n_pad:
            in_specs.append(pl.BlockSpec((br, e.shape[1]), lambda i: (i, 0)))
        else:
            in_specs.append(pl.BlockSpec(e.shape, lambda i: (0, 0)))
    out_shape = [jax.ShapeDtypeStruct((n_pad, 128), jnp.float32)
                 for _ in range(n_out)]
    out_specs = [pl.BlockSpec((br, 128), lambda i: (i, 0)) for _ in range(n_out)]
    res = pl.pallas_call(
        body,
        grid=(grid_n,),
        in_specs=in_specs,
        out_specs=out_specs if n_out > 1 else out_specs[0],
        out_shape=out_shape if n_out > 1 else out_shape[0],
    )(partials, *extras)
    return res


def kernel(pos, norm, edge_index, W1, u1, c1, b1, W2, u2, c2, b2,
           W3, u3, c3, b3, W4, u4, c4, b4, Wl, bl):
    n = pos.shape[0]
    e = edge_index.shape[1]
    f32 = jnp.float32
    n_pad = ((n + 16) + 127) // 128 * 128

    x0 = jnp.concatenate(
        [_bf_rne(pos), _bf_rne(norm), jnp.ones((n, 1), f32), jnp.zeros((n, 9), f32)],
        axis=1)

    n_ch = (e + CH - 1) // CH
    n_ch_p = (n_ch + NW - 1) // NW * NW
    ep = n_ch_p * CH
    pad = ep - e
    src = edge_index[0]
    dst = edge_index[1]
    if pad:
        ar = jnp.arange(pad, dtype=jnp.int32)
        src = jnp.concatenate([src, (ar * 97) % n])
        dst = jnp.concatenate([dst, n + (ar % 16)])
    srcr = src.reshape(NW, n_ch_p // NW, CH)
    dstr = dst.reshape(NW, n_ch_p // NW, CH)

    w1p = jnp.zeros((16, 128), f32).at[:6, :].set(_bf_rne(W1))
    e6 = jnp.zeros((16, 128), f32).at[6, :].set(1.0)
    p1 = _sc_segment_sum(x0, srcr, dstr, n_pad)
    x1, inv = _tc_call(_l1_body, p1, [w1p, e6, b1.reshape(1, 128)], 2)

    x2 = _tc_call(_lmid_body, _sc_segment_sum(x1, srcr, dstr, n_pad),
                  [inv, _bf_rne(W2), b2.reshape(1, 128)], 1)
    x3 = _tc_call(_lmid_body, _sc_segment_sum(x2, srcr, dstr, n_pad),
                  [inv, _bf_rne(W3), b3.reshape(1, 128)], 1)

    wlp = jnp.zeros((128, 128), f32).at[:, :3].set(_bf_rne(Wl))
    blp = jnp.zeros((1, 128), f32).at[0, :3].set(bl)
    out = _tc_call(_l4_body, _sc_segment_sum(x3, srcr, dstr, n_pad),
                   [inv, _bf_rne(W4), b4.reshape(1, 128), wlp, blp], 1)
    return out[:n, :3]


# speedup vs baseline: 18.5534x; 1.3105x over previous
import functools

import jax
import jax.numpy as jnp
from jax import lax
from jax.experimental import pallas as pl
from jax.experimental.pallas import tpu as pltpu
from jax.experimental.pallas import tpu_sc as plsc

NC = 2
NS = 16
NW = NC * NS
CH = 128


@functools.partial(jax.jit, static_argnums=(2,))
def _sc_segment_sum(x, idxr, n_pad):
    ncw = idxr.shape[1]
    d = x.shape[1]
    rpt = n_pad // NS

    mesh = plsc.VectorSubcoreMesh(core_axis_name="c", subcore_axis_name="s",
                                  num_cores=NC, num_subcores=NS)

    @functools.partial(
        pl.kernel,
        out_type=jax.ShapeDtypeStruct((NC, n_pad, d), jnp.float32),
        mesh=mesh,
        scratch_types=[
            pltpu.VMEM((2, CH), jnp.int32),
            pltpu.VMEM((2, CH), jnp.int32),
            pltpu.VMEM((CH, d), jnp.float32),
            pltpu.VMEM((CH, d), jnp.float32),
            pltpu.VMEM_SHARED((n_pad, d), jnp.float32),
            pltpu.SemaphoreType.DMA,
            pltpu.SemaphoreType.DMA,
        ],
        compiler_params=pltpu.CompilerParams(
            use_tc_tiling_on_sc=False if d < 128 else None),
    )
    def seg(x_hbm, idx_hbm, out_hbm, idx_a, idx_b, rows_v, rows_w,
            acc_sh, sem, sem_b):
        c = lax.axis_index("c")
        s = lax.axis_index("s")
        w = c * NS + s

        zv = jnp.zeros((16,), jnp.float32)

        def zrow(i, _):
            for j in range(d // 16):
                rows_v[i, pl.ds(j * 16, 16)] = zv
            return 0

        lax.fori_loop(0, CH, zrow, 0)
        n_full = rpt // CH
        rem = rpt - n_full * CH
        for k in range(n_full):
            pltpu.sync_copy(rows_v, acc_sh.at[pl.ds(s * rpt + k * CH, CH)])
        if rem:
            pltpu.sync_copy(rows_v.at[pl.ds(0, rem)],
                            acc_sh.at[pl.ds(s * rpt + n_full * CH, rem)])
        plsc.subcore_barrier()

        pltpu.sync_copy(idx_hbm.at[w, 0], idx_a)
        pltpu.async_copy(x_hbm.at[idx_a.at[0]], rows_v, sem)

        def body(k, _):
            j0 = 2 * k
            pltpu.sync_copy(idx_hbm.at[w, j0 + 1], idx_b)
            pltpu.async_copy(x_hbm.at[idx_b.at[0]], rows_w, sem_b)
            pltpu.make_async_copy(x_hbm.at[idx_a.at[0]], rows_v, sem).wait()
            pltpu.sync_copy(rows_v, acc_sh.at[idx_a.at[1]], add=True)

            @pl.when(j0 + 2 < ncw)
            def _():
                pltpu.sync_copy(idx_hbm.at[w, j0 + 2], idx_a)
                pltpu.async_copy(x_hbm.at[idx_a.at[0]], rows_v, sem)

            pltpu.make_async_copy(x_hbm.at[idx_b.at[0]], rows_w, sem_b).wait()
            pltpu.sync_copy(rows_w, acc_sh.at[idx_b.at[1]], add=True)
            return 0

        lax.fori_loop(0, ncw // 2, body, 0)
        plsc.subcore_barrier()

        pltpu.sync_copy(acc_sh.at[pl.ds(s * rpt, rpt)],
                        out_hbm.at[c, pl.ds(s * rpt, rpt)])

    return seg(x, idxr)


_DOT = dict(preferred_element_type=jnp.float32, precision=lax.Precision.HIGHEST)


def _bf_rne(x):
    u = lax.bitcast_convert_type(x, jnp.uint32)
    r = ((u >> 16) & 1) + jnp.uint32(0x7FFF)
    return lax.bitcast_convert_type((u + r) & jnp.uint32(0xFFFF0000),
                                    jnp.float32)


def _l1_body(p_ref, w_ref, e6_ref, b_ref, x_ref, inv_ref):
    p = p_ref[0] + p_ref[1]
    y = jnp.dot(p, w_ref[...], **_DOT)
    cnt = jnp.dot(p, e6_ref[...], **_DOT)
    inv = 1.0 / jnp.maximum(cnt, 1.0)
    x_ref[...] = _bf_rne(jnp.maximum(y * inv + b_ref[...], 0.0))
    inv_ref[...] = inv


def _lmid_body(p_ref, inv_ref, w_ref, b_ref, x_ref):
    agg = (p_ref[0] + p_ref[1]) * inv_ref[...]
    y = jnp.dot(agg, w_ref[...], **_DOT) + b_ref[...]
    x_ref[...] = _bf_rne(jnp.maximum(y, 0.0))


def _l4_body(p_ref, inv_ref, w_ref, b_ref, wl_ref, bl_ref, o_ref):
    agg = (p_ref[0] + p_ref[1]) * inv_ref[...]
    x4 = jnp.maximum(jnp.dot(agg, w_ref[...], **_DOT) + b_ref[...], 0.0)
    o_ref[...] = jnp.dot(_bf_rne(x4), wl_ref[...], **_DOT) + bl_ref[...]


def _tc_call(body, partials, extras, n_out, grid_n=4):
    n_pad, din = partials.shape[1], partials.shape[2]
    br = n_pad // grid_n
    in_specs = [pl.BlockSpec((NC, br, din), lambda i: (0, i, 0))]
    for e in extras:
        if e.shape[0] == n_pad:
            in_specs.append(pl.BlockSpec((br, e.shape[1]), lambda i: (i, 0)))
        else:
            in_specs.append(pl.BlockSpec(e.shape, lambda i: (0, 0)))
    out_shape = [jax.ShapeDtypeStruct((n_pad, 128), jnp.float32)
                 for _ in range(n_out)]
    out_specs = [pl.BlockSpec((br, 128), lambda i: (i, 0)) for _ in range(n_out)]
    res = pl.pallas_call(
        body,
        grid=(grid_n,),
        in_specs=in_specs,
        out_specs=out_specs if n_out > 1 else out_specs[0],
        out_shape=out_shape if n_out > 1 else out_shape[0],
    )(partials, *extras)
    return res


def kernel(pos, norm, edge_index, W1, u1, c1, b1, W2, u2, c2, b2,
           W3, u3, c3, b3, W4, u4, c4, b4, Wl, bl):
    n = pos.shape[0]
    e = edge_index.shape[1]
    f32 = jnp.float32
    n_pad = ((n + 16) + 127) // 128 * 128

    x0 = jnp.concatenate(
        [_bf_rne(pos), _bf_rne(norm), jnp.ones((n, 1), f32), jnp.zeros((n, 9), f32)],
        axis=1)

    n_ch = (e + CH - 1) // CH
    n_ch_p = (n_ch + 2 * NW - 1) // (2 * NW) * (2 * NW)
    ep = n_ch_p * CH
    pad = ep - e
    src = edge_index[0]
    dst = edge_index[1]
    if pad:
        ar = jnp.arange(pad, dtype=jnp.int32)
        src = jnp.concatenate([src, (ar * 97) % n])
        dst = jnp.concatenate([dst, n + (ar % 16)])
    ncw = n_ch_p // NW
    idxr = jnp.stack([src.reshape(NW, ncw, CH), dst.reshape(NW, ncw, CH)],
                     axis=2)

    w1p = jnp.zeros((16, 128), f32).at[:6, :].set(_bf_rne(W1))
    e6 = jnp.zeros((16, 128), f32).at[6, :].set(1.0)
    p1 = _sc_segment_sum(x0, idxr, n_pad)
    x1, inv = _tc_call(_l1_body, p1, [w1p, e6, b1.reshape(1, 128)], 2)

    x2 = _tc_call(_lmid_body, _sc_segment_sum(x1, idxr, n_pad),
                  [inv, _bf_rne(W2), b2.reshape(1, 128)], 1)
    x3 = _tc_call(_lmid_body, _sc_segment_sum(x2, idxr, n_pad),
                  [inv, _bf_rne(W3), b3.reshape(1, 128)], 1)

    wlp = jnp.zeros((128, 128), f32).at[:, :3].set(_bf_rne(Wl))
    blp = jnp.zeros((1, 128), f32).at[0, :3].set(bl)
    out = _tc_call(_l4_body, _sc_segment_sum(x3, idxr, n_pad),
                   [inv, _bf_rne(W4), b4.reshape(1, 128), wlp, blp], 1)
    return out[:n, :3]
